# Initial kernel scaffold; baseline (speedup 1.0000x reference)
import jax
import jax.numpy as jnp
from jax.experimental import pallas as pl


def kernel(features, edge_index, W1, b1, W2, b2, Wp1, bp1, Wp2, bp2, Wv1, bv1, Wv2, bv2):
    raise NotImplementedError("write your pallas kernel here")


# speedup vs baseline: 59.4718x; 59.4718x over previous
import functools

import jax
import jax.numpy as jnp
from jax import lax
from jax.experimental import pallas as pl
from jax.experimental.pallas import tpu as pltpu
from jax.experimental.pallas import tpu_sc as plsc

N = 10000
NP = 10240
F_IN = 128
H = 32
B = 4
E = 320000
BH = B * H
NA = 18

NC = 2
NS = 16
NW = NC * NS
EW = E // NW
K = 80
CH = EW // K
IB = 25
NB = CH // IB
RPT = NP // NS
ZR = 64

_mesh = plsc.VectorSubcoreMesh(core_axis_name="c", subcore_axis_name="s")


@functools.partial(
    pl.kernel,
    out_type=jax.ShapeDtypeStruct((NW * NP,), jnp.float32),
    mesh=_mesh,
    scratch_types=[
        pltpu.VMEM((CH, K), jnp.int32),
        pltpu.VMEM((NP,), jnp.float32),
    ],
    compiler_params=pltpu.CompilerParams(needs_layout_passes=False),
)
def _deg(dst_hbm, out_hbm, dst_v, deg_l):
    c = lax.axis_index("c")
    s = lax.axis_index("s")
    w = s * NC + c

    def zero_body(i, _):
        deg_l[pl.ds(i * 16, 16)] = jnp.zeros((16,), jnp.float32)
        return 0

    lax.fori_loop(0, NP // 16, zero_body, 0)
    pltpu.sync_copy(dst_hbm.at[w], dst_v)
    ones = jnp.ones((16,), jnp.float32)

    def body(j, _):
        for t in range(K // 16):
            idx = dst_v[j, pl.ds(t * 16, 16)]
            plsc.addupdate_scatter(deg_l, [idx], ones)
        return 0

    lax.fori_loop(0, CH, body, 0)
    pltpu.sync_copy(deg_l, out_hbm.at[pl.ds(w * NP, NP)])


@functools.partial(
    pl.kernel,
    out_type=jax.ShapeDtypeStruct((NC, NP, BH), jnp.float32),
    mesh=_mesh,
    scratch_types=[
        pltpu.VMEM((IB, K), jnp.int32),
        pltpu.VMEM((IB, K), jnp.int32),
        pltpu.VMEM((K, BH), jnp.float32),
        pltpu.VMEM((ZR, BH), jnp.float32),
        pltpu.VMEM_SHARED((NP, BH), jnp.float32),
        pltpu.SemaphoreType.DMA,
    ],
    compiler_params=pltpu.CompilerParams(needs_layout_passes=False),
)
def _mp(src_hbm, dst_hbm, table_hbm, out_hbm, src_v, dst_v, rows, zbuf, acc,
        gsem):
    c = lax.axis_index("c")
    s = lax.axis_index("s")
    w = s * NC + c

    def zero_body(i, _):
        for t in range(BH // 16):
            zbuf[i, pl.ds(t * 16, 16)] = jnp.zeros((16,), jnp.float32)
        return 0

    lax.fori_loop(0, ZR, zero_body, 0)
    base = s * RPT
    for i in range(RPT // ZR):
        pltpu.sync_copy(zbuf, acc.at[pl.ds(base + i * ZR, ZR)])
    plsc.subcore_barrier()

    for blk in range(NB):
        pltpu.sync_copy(src_hbm.at[w, blk], src_v)
        pltpu.sync_copy(dst_hbm.at[w, blk], dst_v)

        def body(j, _):
            pltpu.async_copy(table_hbm.at[src_v.at[j]], rows, gsem).wait()
            pltpu.sync_copy(rows, acc.at[dst_v.at[j]], add=True)
            return 0

        lax.fori_loop(0, IB, body, 0)
    plsc.subcore_barrier()

    for i in range(RPT // ZR):
        pltpu.sync_copy(acc.at[pl.ds(base + i * ZR, ZR)], zbuf)
        pltpu.sync_copy(zbuf, out_hbm.at[c, pl.ds(base + i * ZR, ZR)])


_BN = 1024


def _prep_body(ft_ref, w_ref, dinv_ref, out_ref):
    h = jnp.dot(ft_ref[...], w_ref[...], preferred_element_type=jnp.float32)
    out_ref[...] = dinv_ref[...] * h


def _mid_body(p_ref, h_ref, dinv_ref, b_ref, w_ref, out_ref):
    x = p_ref[0] + p_ref[1] + h_ref[...]
    x = jnp.maximum(dinv_ref[...] * x + b_ref[...], 0.0)
    out_ref[...] = dinv_ref[...] * jnp.dot(
        x, w_ref[...], preferred_element_type=jnp.float32)


def _final_body(p_ref, h_ref, dinv_ref, b_ref, out_ref):
    x = p_ref[0] + p_ref[1] + h_ref[...]
    out_ref[...] = jnp.maximum(dinv_ref[...] * x + b_ref[...], 0.0)


_BK = 2560
_KS = (N * H) // _BK


def _heads_body(conv_ref, wp1_ref, bp1_ref, wp2_ref, bp2_ref,
                wv1_ref, bv1_ref, wv2_ref, bv2_ref,
                pol_ref, val_ref, accp, accv):
    k = pl.program_id(0)

    @pl.when(k == 0)
    def _():
        accp[...] = jnp.zeros_like(accp)
        accv[...] = jnp.zeros_like(accv)

    cblk = conv_ref[...]
    accp[...] += jnp.dot(cblk, wp1_ref[...], preferred_element_type=jnp.float32)
    accv[...] += jnp.dot(cblk, wv1_ref[...], preferred_element_type=jnp.float32)

    @pl.when(k == _KS - 1)
    def _():
        hp = jnp.maximum(accp[...] + bp1_ref[...], 0.0)
        pol_ref[...] = jnp.dot(
            hp, wp2_ref[...], preferred_element_type=jnp.float32) + bp2_ref[...]
        hv = jnp.maximum(accv[...] + bv1_ref[...], 0.0)
        val_ref[...] = jnp.dot(
            hv, wv2_ref[...], preferred_element_type=jnp.float32) + bv2_ref[...]


def kernel(features, edge_index, W1, b1, W2, b2,
           Wp1, bp1, Wp2, bp2, Wv1, bv1, Wv2, bv2):
    f32 = jnp.float32
    src4d = edge_index[0].reshape(NW, NB, IB, K)
    dst4d = edge_index[1].reshape(NW, NB, IB, K)
    dst3d = edge_index[1].reshape(NW, CH, K)

    deg_part = _deg(dst3d).reshape(NW, NP)
    deg = 1.0 + jnp.sum(deg_part, axis=0)
    dinv = lax.rsqrt(deg)
    dinv_b = jnp.broadcast_to(dinv[:, None], (NP, BH))

    ft = jnp.transpose(features, (1, 0, 2)).reshape(N, B * F_IN)
    ft = jnp.pad(ft, ((0, NP - N), (0, 0)))
    W1bd = jnp.kron(jnp.eye(B, dtype=f32), W1)
    W2bd = jnp.kron(jnp.eye(B, dtype=f32), W2)
    b1t = jnp.tile(b1, B).reshape(1, BH)
    b2t = jnp.tile(b2, B).reshape(1, BH)

    h1 = pl.pallas_call(
        _prep_body,
        grid=(NP // _BN,),
        in_specs=[
            pl.BlockSpec((_BN, B * F_IN), lambda i: (i, 0)),
            pl.BlockSpec((B * F_IN, BH), lambda i: (0, 0)),
            pl.BlockSpec((_BN, BH), lambda i: (i, 0)),
        ],
        out_specs=pl.BlockSpec((_BN, BH), lambda i: (i, 0)),
        out_shape=jax.ShapeDtypeStruct((NP, BH), f32),
    )(ft, W1bd, dinv_b)

    p1 = _mp(src4d, dst4d, h1)

    h2 = pl.pallas_call(
        _mid_body,
        grid=(NP // _BN,),
        in_specs=[
            pl.BlockSpec((NC, _BN, BH), lambda i: (0, i, 0)),
            pl.BlockSpec((_BN, BH), lambda i: (i, 0)),
            pl.BlockSpec((_BN, BH), lambda i: (i, 0)),
            pl.BlockSpec((1, BH), lambda i: (0, 0)),
            pl.BlockSpec((BH, BH), lambda i: (0, 0)),
        ],
        out_specs=pl.BlockSpec((_BN, BH), lambda i: (i, 0)),
        out_shape=jax.ShapeDtypeStruct((NP, BH), f32),
    )(p1, h1, dinv_b, b1t, W2bd)

    p2 = _mp(src4d, dst4d, h2)

    x2 = pl.pallas_call(
        _final_body,
        grid=(NP // _BN,),
        in_specs=[
            pl.BlockSpec((NC, _BN, BH), lambda i: (0, i, 0)),
            pl.BlockSpec((_BN, BH), lambda i: (i, 0)),
            pl.BlockSpec((_BN, BH), lambda i: (i, 0)),
            pl.BlockSpec((1, BH), lambda i: (0, 0)),
        ],
        out_specs=pl.BlockSpec((_BN, BH), lambda i: (i, 0)),
        out_shape=jax.ShapeDtypeStruct((NP, BH), f32),
    )(p2, h2, dinv_b, b2t)

    conv = jnp.transpose(x2[:N].reshape(N, B, H), (1, 0, 2)).reshape(B, N * H)

    Wp2p = jnp.pad(Wp2, ((0, 0), (0, 128 - NA)))
    bp2p = jnp.pad(bp2, (0, 128 - NA)).reshape(1, 128)
    Wv2p = jnp.pad(Wv2, ((0, 0), (0, 128 - 1)))
    bv2p = jnp.pad(bv2, (0, 128 - 1)).reshape(1, 128)

    pol_p, val_p = pl.pallas_call(
        _heads_body,
        grid=(_KS,),
        in_specs=[
            pl.BlockSpec((B, _BK), lambda k: (0, k)),
            pl.BlockSpec((_BK, 512), lambda k: (k, 0)),
            pl.BlockSpec((1, 512), lambda k: (0, 0)),
            pl.BlockSpec((512, 128), lambda k: (0, 0)),
            pl.BlockSpec((1, 128), lambda k: (0, 0)),
            pl.BlockSpec((_BK, 512), lambda k: (k, 0)),
            pl.BlockSpec((1, 512), lambda k: (0, 0)),
            pl.BlockSpec((512, 128), lambda k: (0, 0)),
            pl.BlockSpec((1, 128), lambda k: (0, 0)),
        ],
        out_specs=[
            pl.BlockSpec((B, 128), lambda k: (0, 0)),
            pl.BlockSpec((B, 128), lambda k: (0, 0)),
        ],
        out_shape=[
            jax.ShapeDtypeStruct((B, 128), f32),
            jax.ShapeDtypeStruct((B, 128), f32),
        ],
        scratch_shapes=[
            pltpu.VMEM((B, 512), f32),
            pltpu.VMEM((B, 512), f32),
        ],
        compiler_params=pltpu.CompilerParams(
            dimension_semantics=("arbitrary",)),
    )(conv, Wp1, bp1.reshape(1, 512), Wp2p, bp2p,
      Wv1, bv1.reshape(1, 512), Wv2p, bv2p)

    policy = pol_p[:, :NA]
    value = val_p[:, :1]
    return (policy, value)


# speedup vs baseline: 68.8331x; 1.1574x over previous
import functools

import jax
import jax.numpy as jnp
from jax import lax
from jax.experimental import pallas as pl
from jax.experimental.pallas import tpu as pltpu
from jax.experimental.pallas import tpu_sc as plsc

N = 10000
NP = 10240
F_IN = 128
H = 32
B = 4
E = 320000
BH = B * H
NA = 18

NC = 2
NS = 16
NW = NC * NS
EW = E // NW
KD = 80
CHD = EW // KD
K = 100
IB = 20
NB = EW // (IB * K)
RPT = NP // NS
ZR = 32

_mesh = plsc.VectorSubcoreMesh(core_axis_name="c", subcore_axis_name="s")


@functools.partial(
    pl.kernel,
    out_type=jax.ShapeDtypeStruct((NW * NP,), jnp.float32),
    mesh=_mesh,
    scratch_types=[
        pltpu.VMEM((CHD, KD), jnp.int32),
        pltpu.VMEM((NP,), jnp.float32),
    ],
    compiler_params=pltpu.CompilerParams(needs_layout_passes=False),
)
def _deg(dst_hbm, out_hbm, dst_v, deg_l):
    c = lax.axis_index("c")
    s = lax.axis_index("s")
    w = s * NC + c

    def zero_body(i, _):
        deg_l[pl.ds(i * 16, 16)] = jnp.zeros((16,), jnp.float32)
        return 0

    lax.fori_loop(0, NP // 16, zero_body, 0)
    pltpu.sync_copy(dst_hbm.at[w], dst_v)
    ones = jnp.ones((16,), jnp.float32)

    def body(j, _):
        for t in range(KD // 16):
            idx = dst_v[j, pl.ds(t * 16, 16)]
            plsc.addupdate_scatter(deg_l, [idx], ones)
        return 0

    lax.fori_loop(0, CHD, body, 0)
    pltpu.sync_copy(deg_l, out_hbm.at[pl.ds(w * NP, NP)])


@functools.partial(
    pl.kernel,
    out_type=jax.ShapeDtypeStruct((NC, NP, BH), jnp.float32),
    mesh=_mesh,
    scratch_types=[
        pltpu.VMEM((IB, K), jnp.int32),
        pltpu.VMEM((IB, K), jnp.int32),
        pltpu.VMEM((K, BH), jnp.float32),
        pltpu.VMEM((K, BH), jnp.float32),
        pltpu.VMEM((ZR, BH), jnp.float32),
        pltpu.VMEM_SHARED((NP, BH), jnp.float32),
        pltpu.SemaphoreType.DMA,
        pltpu.SemaphoreType.DMA,
    ],
    compiler_params=pltpu.CompilerParams(needs_layout_passes=False),
)
def _mp(src_hbm, dst_hbm, table_hbm, out_hbm, src_v, dst_v, rows0, rows1,
        zbuf, acc, sem0, sem1):
    c = lax.axis_index("c")
    s = lax.axis_index("s")
    w = s * NC + c

    def zero_body(i, _):
        for t in range(BH // 16):
            zbuf[i, pl.ds(t * 16, 16)] = jnp.zeros((16,), jnp.float32)
        return 0

    lax.fori_loop(0, ZR, zero_body, 0)
    base = s * RPT
    for i in range(RPT // ZR):
        pltpu.sync_copy(zbuf, acc.at[pl.ds(base + i * ZR, ZR)])
    plsc.subcore_barrier()

    for blk in range(NB):
        pltpu.sync_copy(src_hbm.at[w, blk], src_v)
        pltpu.sync_copy(dst_hbm.at[w, blk], dst_v)
        pltpu.async_copy(table_hbm.at[src_v.at[0]], rows0, sem0)

        def pair(jj, _):
            j0 = 2 * jj
            pltpu.make_async_copy(
                table_hbm.at[src_v.at[j0]], rows0, sem0).wait()
            pltpu.async_copy(table_hbm.at[src_v.at[j0 + 1]], rows1, sem1)
            pltpu.sync_copy(rows0, acc.at[dst_v.at[j0]], add=True)
            pltpu.make_async_copy(
                table_hbm.at[src_v.at[j0 + 1]], rows1, sem1).wait()
            pltpu.async_copy(table_hbm.at[src_v.at[j0 + 2]], rows0, sem0)
            pltpu.sync_copy(rows1, acc.at[dst_v.at[j0 + 1]], add=True)
            return 0

        lax.fori_loop(0, IB // 2 - 1, pair, 0)
        pltpu.make_async_copy(table_hbm.at[src_v.at[IB - 2]], rows0,
                              sem0).wait()
        pltpu.async_copy(table_hbm.at[src_v.at[IB - 1]], rows1, sem1)
        pltpu.sync_copy(rows0, acc.at[dst_v.at[IB - 2]], add=True)
        pltpu.make_async_copy(table_hbm.at[src_v.at[IB - 1]], rows1,
                              sem1).wait()
        pltpu.sync_copy(rows1, acc.at[dst_v.at[IB - 1]], add=True)
    plsc.subcore_barrier()

    for i in range(RPT // ZR):
        pltpu.sync_copy(acc.at[pl.ds(base + i * ZR, ZR)], zbuf)
        pltpu.sync_copy(zbuf, out_hbm.at[c, pl.ds(base + i * ZR, ZR)])


_BN = 1024


def _prep_body(ft_ref, w_ref, dinv_ref, out_ref):
    h = jnp.dot(ft_ref[...], w_ref[...], preferred_element_type=jnp.float32)
    out_ref[...] = dinv_ref[...] * h


def _mid_body(p_ref, h_ref, dinv_ref, b_ref, w_ref, out_ref):
    x = p_ref[0] + p_ref[1] + h_ref[...]
    x = jnp.maximum(dinv_ref[...] * x + b_ref[...], 0.0)
    out_ref[...] = dinv_ref[...] * jnp.dot(
        x, w_ref[...], preferred_element_type=jnp.float32)


def _final_body(p_ref, h_ref, dinv_ref, b_ref, out_ref):
    x = p_ref[0] + p_ref[1] + h_ref[...]
    out_ref[...] = jnp.maximum(dinv_ref[...] * x + b_ref[...], 0.0)


_BK = 2560
_KS = (N * H) // _BK


def _heads_body(conv_ref, wp1_ref, bp1_ref, wp2_ref, bp2_ref,
                wv1_ref, bv1_ref, wv2_ref, bv2_ref,
                pol_ref, val_ref, accp, accv):
    k = pl.program_id(0)

    @pl.when(k == 0)
    def _():
        accp[...] = jnp.zeros_like(accp)
        accv[...] = jnp.zeros_like(accv)

    cblk = conv_ref[...]
    accp[...] += jnp.dot(cblk, wp1_ref[...], preferred_element_type=jnp.float32)
    accv[...] += jnp.dot(cblk, wv1_ref[...], preferred_element_type=jnp.float32)

    @pl.when(k == _KS - 1)
    def _():
        hp = jnp.maximum(accp[...] + bp1_ref[...], 0.0)
        pol_ref[...] = jnp.dot(
            hp, wp2_ref[...], preferred_element_type=jnp.float32) + bp2_ref[...]
        hv = jnp.maximum(accv[...] + bv1_ref[...], 0.0)
        val_ref[...] = jnp.dot(
            hv, wv2_ref[...], preferred_element_type=jnp.float32) + bv2_ref[...]


def kernel(features, edge_index, W1, b1, W2, b2,
           Wp1, bp1, Wp2, bp2, Wv1, bv1, Wv2, bv2):
    f32 = jnp.float32
    src4d = edge_index[0].reshape(NW, NB, IB, K)
    dst4d = edge_index[1].reshape(NW, NB, IB, K)
    dst3d = edge_index[1].reshape(NW, CHD, KD)

    deg_part = _deg(dst3d).reshape(NW, NP)
    deg = 1.0 + jnp.sum(deg_part, axis=0)
    dinv = lax.rsqrt(deg)
    dinv_b = jnp.broadcast_to(dinv[:, None], (NP, BH))

    ft = jnp.transpose(features, (1, 0, 2)).reshape(N, B * F_IN)
    ft = jnp.pad(ft, ((0, NP - N), (0, 0)))
    W1bd = jnp.kron(jnp.eye(B, dtype=f32), W1)
    W2bd = jnp.kron(jnp.eye(B, dtype=f32), W2)
    b1t = jnp.tile(b1, B).reshape(1, BH)
    b2t = jnp.tile(b2, B).reshape(1, BH)

    h1 = pl.pallas_call(
        _prep_body,
        grid=(NP // _BN,),
        in_specs=[
            pl.BlockSpec((_BN, B * F_IN), lambda i: (i, 0)),
            pl.BlockSpec((B * F_IN, BH), lambda i: (0, 0)),
            pl.BlockSpec((_BN, BH), lambda i: (i, 0)),
        ],
        out_specs=pl.BlockSpec((_BN, BH), lambda i: (i, 0)),
        out_shape=jax.ShapeDtypeStruct((NP, BH), f32),
    )(ft, W1bd, dinv_b)

    p1 = _mp(src4d, dst4d, h1)

    h2 = pl.pallas_call(
        _mid_body,
        grid=(NP // _BN,),
        in_specs=[
            pl.BlockSpec((NC, _BN, BH), lambda i: (0, i, 0)),
            pl.BlockSpec((_BN, BH), lambda i: (i, 0)),
            pl.BlockSpec((_BN, BH), lambda i: (i, 0)),
            pl.BlockSpec((1, BH), lambda i: (0, 0)),
            pl.BlockSpec((BH, BH), lambda i: (0, 0)),
        ],
        out_specs=pl.BlockSpec((_BN, BH), lambda i: (i, 0)),
        out_shape=jax.ShapeDtypeStruct((NP, BH), f32),
    )(p1, h1, dinv_b, b1t, W2bd)

    p2 = _mp(src4d, dst4d, h2)

    x2 = pl.pallas_call(
        _final_body,
        grid=(NP // _BN,),
        in_specs=[
            pl.BlockSpec((NC, _BN, BH), lambda i: (0, i, 0)),
            pl.BlockSpec((_BN, BH), lambda i: (i, 0)),
            pl.BlockSpec((_BN, BH), lambda i: (i, 0)),
            pl.BlockSpec((1, BH), lambda i: (0, 0)),
        ],
        out_specs=pl.BlockSpec((_BN, BH), lambda i: (i, 0)),
        out_shape=jax.ShapeDtypeStruct((NP, BH), f32),
    )(p2, h2, dinv_b, b2t)

    conv = jnp.transpose(x2[:N].reshape(N, B, H), (1, 0, 2)).reshape(B, N * H)

    Wp2p = jnp.pad(Wp2, ((0, 0), (0, 128 - NA)))
    bp2p = jnp.pad(bp2, (0, 128 - NA)).reshape(1, 128)
    Wv2p = jnp.pad(Wv2, ((0, 0), (0, 128 - 1)))
    bv2p = jnp.pad(bv2, (0, 128 - 1)).reshape(1, 128)

    pol_p, val_p = pl.pallas_call(
        _heads_body,
        grid=(_KS,),
        in_specs=[
            pl.BlockSpec((B, _BK), lambda k: (0, k)),
            pl.BlockSpec((_BK, 512), lambda k: (k, 0)),
            pl.BlockSpec((1, 512), lambda k: (0, 0)),
            pl.BlockSpec((512, 128), lambda k: (0, 0)),
            pl.BlockSpec((1, 128), lambda k: (0, 0)),
            pl.BlockSpec((_BK, 512), lambda k: (k, 0)),
            pl.BlockSpec((1, 512), lambda k: (0, 0)),
            pl.BlockSpec((512, 128), lambda k: (0, 0)),
            pl.BlockSpec((1, 128), lambda k: (0, 0)),
        ],
        out_specs=[
            pl.BlockSpec((B, 128), lambda k: (0, 0)),
            pl.BlockSpec((B, 128), lambda k: (0, 0)),
        ],
        out_shape=[
            jax.ShapeDtypeStruct((B, 128), f32),
            jax.ShapeDtypeStruct((B, 128), f32),
        ],
        scratch_shapes=[
            pltpu.VMEM((B, 512), f32),
            pltpu.VMEM((B, 512), f32),
        ],
        compiler_params=pltpu.CompilerParams(
            dimension_semantics=("arbitrary",)),
    )(conv, Wp1, bp1.reshape(1, 512), Wp2p, bp2p,
      Wv1, bv1.reshape(1, 512), Wv2p, bv2p)

    policy = pol_p[:, :NA]
    value = val_p[:, :1]
    return (policy, value)


# speedup vs baseline: 69.2431x; 1.0060x over previous
import functools

import jax
import jax.numpy as jnp
from jax import lax
from jax.experimental import pallas as pl
from jax.experimental.pallas import tpu as pltpu
from jax.experimental.pallas import tpu_sc as plsc

N = 10000
NP = 10240
F_IN = 128
H = 32
B = 4
E = 320000
BH = B * H
NA = 18

NC = 2
NS = 16
NW = NC * NS
EW = E // NW
KD = 80
CHD = EW // KD
K = 100
IB = 20
NB = EW // (IB * K)
RPT = NP // NS
ZR = 32

_mesh = plsc.VectorSubcoreMesh(core_axis_name="c", subcore_axis_name="s")


@functools.partial(
    pl.kernel,
    out_type=jax.ShapeDtypeStruct((NW * NP,), jnp.float32),
    mesh=_mesh,
    scratch_types=[
        pltpu.VMEM((CHD, KD), jnp.int32),
        pltpu.VMEM((NP,), jnp.float32),
    ],
    compiler_params=pltpu.CompilerParams(needs_layout_passes=False),
)
def _deg(dst_hbm, out_hbm, dst_v, deg_l):
    c = lax.axis_index("c")
    s = lax.axis_index("s")
    w = s * NC + c

    def zero_body(i, _):
        deg_l[pl.ds(i * 16, 16)] = jnp.zeros((16,), jnp.float32)
        return 0

    lax.fori_loop(0, NP // 16, zero_body, 0)
    pltpu.sync_copy(dst_hbm.at[w], dst_v)
    ones = jnp.ones((16,), jnp.float32)

    def body(j, _):
        for t in range(KD // 16):
            idx = dst_v[j, pl.ds(t * 16, 16)]
            plsc.addupdate_scatter(deg_l, [idx], ones)
        return 0

    lax.fori_loop(0, CHD, body, 0)
    pltpu.sync_copy(deg_l, out_hbm.at[pl.ds(w * NP, NP)])


@functools.partial(
    pl.kernel,
    out_type=jax.ShapeDtypeStruct((NC, NP, BH), jnp.float32),
    mesh=_mesh,
    scratch_types=[
        pltpu.VMEM((IB, K), jnp.int32),
        pltpu.VMEM((IB, K), jnp.int32),
        pltpu.VMEM((K, BH), jnp.float32),
        pltpu.VMEM((K, BH), jnp.float32),
        pltpu.VMEM((ZR, BH), jnp.float32),
        pltpu.VMEM_SHARED((NP, BH), jnp.float32),
        pltpu.SemaphoreType.DMA,
        pltpu.SemaphoreType.DMA,
    ],
    compiler_params=pltpu.CompilerParams(needs_layout_passes=False),
)
def _mp(src_hbm, dst_hbm, table_hbm, out_hbm, src_v, dst_v, rows0, rows1,
        zbuf, acc, sem0, sem1):
    c = lax.axis_index("c")
    s = lax.axis_index("s")
    w = s * NC + c

    def zero_body(i, _):
        for t in range(BH // 16):
            zbuf[i, pl.ds(t * 16, 16)] = jnp.zeros((16,), jnp.float32)
        return 0

    lax.fori_loop(0, ZR, zero_body, 0)
    base = s * RPT
    for i in range(RPT // ZR):
        pltpu.sync_copy(zbuf, acc.at[pl.ds(base + i * ZR, ZR)])
    plsc.subcore_barrier()

    for blk in range(NB):
        pltpu.sync_copy(src_hbm.at[w, blk], src_v)
        pltpu.sync_copy(dst_hbm.at[w, blk], dst_v)
        pltpu.async_copy(table_hbm.at[src_v.at[0]], rows0, sem0)

        def pair(jj, _):
            j0 = 2 * jj
            pltpu.make_async_copy(
                table_hbm.at[src_v.at[j0]], rows0, sem0).wait()
            pltpu.async_copy(table_hbm.at[src_v.at[j0 + 1]], rows1, sem1)
            pltpu.sync_copy(rows0, acc.at[dst_v.at[j0]], add=True)
            pltpu.make_async_copy(
                table_hbm.at[src_v.at[j0 + 1]], rows1, sem1).wait()
            pltpu.async_copy(table_hbm.at[src_v.at[j0 + 2]], rows0, sem0)
            pltpu.sync_copy(rows1, acc.at[dst_v.at[j0 + 1]], add=True)
            return 0

        lax.fori_loop(0, IB // 2 - 1, pair, 0)
        pltpu.make_async_copy(table_hbm.at[src_v.at[IB - 2]], rows0,
                              sem0).wait()
        pltpu.async_copy(table_hbm.at[src_v.at[IB - 1]], rows1, sem1)
        pltpu.sync_copy(rows0, acc.at[dst_v.at[IB - 2]], add=True)
        pltpu.make_async_copy(table_hbm.at[src_v.at[IB - 1]], rows1,
                              sem1).wait()
        pltpu.sync_copy(rows1, acc.at[dst_v.at[IB - 1]], add=True)
    plsc.subcore_barrier()

    pltpu.sync_copy(acc.at[pl.ds(base, RPT)],
                    out_hbm.at[c, pl.ds(base, RPT)])


_BN = 1024


def _prep_body(ft_ref, w_ref, dinv_ref, out_ref):
    h = jnp.dot(ft_ref[...], w_ref[...], preferred_element_type=jnp.float32)
    out_ref[...] = dinv_ref[...] * h


def _mid_body(p_ref, h_ref, dinv_ref, b_ref, w_ref, out_ref):
    x = p_ref[0] + p_ref[1] + h_ref[...]
    x = jnp.maximum(dinv_ref[...] * x + b_ref[...], 0.0)
    out_ref[...] = dinv_ref[...] * jnp.dot(
        x, w_ref[...], preferred_element_type=jnp.float32)


def _final_body(p_ref, h_ref, dinv_ref, b_ref, out_ref):
    x = p_ref[0] + p_ref[1] + h_ref[...]
    out_ref[...] = jnp.maximum(dinv_ref[...] * x + b_ref[...], 0.0)


_BK = 3200
_KS = (N * H) // _BK


def _heads_body(conv_ref, wp1_ref, bp1_ref, wp2_ref, bp2_ref,
                wv1_ref, bv1_ref, wv2_ref, bv2_ref,
                pol_ref, val_ref, accp, accv):
    k = pl.program_id(0)

    @pl.when(k == 0)
    def _():
        accp[...] = jnp.zeros_like(accp)
        accv[...] = jnp.zeros_like(accv)

    cblk = conv_ref[...]
    accp[...] += jnp.dot(cblk, wp1_ref[...], preferred_element_type=jnp.float32)
    accv[...] += jnp.dot(cblk, wv1_ref[...], preferred_element_type=jnp.float32)

    @pl.when(k == _KS - 1)
    def _():
        hp = jnp.maximum(accp[...] + bp1_ref[...], 0.0)
        pol_ref[...] = jnp.dot(
            hp, wp2_ref[...], preferred_element_type=jnp.float32) + bp2_ref[...]
        hv = jnp.maximum(accv[...] + bv1_ref[...], 0.0)
        val_ref[...] = jnp.dot(
            hv, wv2_ref[...], preferred_element_type=jnp.float32) + bv2_ref[...]


def kernel(features, edge_index, W1, b1, W2, b2,
           Wp1, bp1, Wp2, bp2, Wv1, bv1, Wv2, bv2):
    f32 = jnp.float32
    src4d = edge_index[0].reshape(NW, NB, IB, K)
    dst4d = edge_index[1].reshape(NW, NB, IB, K)
    dst3d = edge_index[1].reshape(NW, CHD, KD)

    deg_part = _deg(dst3d).reshape(NW, NP)
    deg = 1.0 + jnp.sum(deg_part, axis=0)
    dinv = lax.rsqrt(deg)
    dinv_b = jnp.broadcast_to(dinv[:, None], (NP, BH))

    ft = jnp.transpose(features, (1, 0, 2)).reshape(N, B * F_IN)
    ft = jnp.pad(ft, ((0, NP - N), (0, 0)))
    W1bd = jnp.kron(jnp.eye(B, dtype=f32), W1)
    W2bd = jnp.kron(jnp.eye(B, dtype=f32), W2)
    b1t = jnp.tile(b1, B).reshape(1, BH)
    b2t = jnp.tile(b2, B).reshape(1, BH)

    h1 = pl.pallas_call(
        _prep_body,
        grid=(NP // _BN,),
        in_specs=[
            pl.BlockSpec((_BN, B * F_IN), lambda i: (i, 0)),
            pl.BlockSpec((B * F_IN, BH), lambda i: (0, 0)),
            pl.BlockSpec((_BN, BH), lambda i: (i, 0)),
        ],
        out_specs=pl.BlockSpec((_BN, BH), lambda i: (i, 0)),
        out_shape=jax.ShapeDtypeStruct((NP, BH), f32),
    )(ft, W1bd, dinv_b)

    p1 = _mp(src4d, dst4d, h1)

    h2 = pl.pallas_call(
        _mid_body,
        grid=(NP // _BN,),
        in_specs=[
            pl.BlockSpec((NC, _BN, BH), lambda i: (0, i, 0)),
            pl.BlockSpec((_BN, BH), lambda i: (i, 0)),
            pl.BlockSpec((_BN, BH), lambda i: (i, 0)),
            pl.BlockSpec((1, BH), lambda i: (0, 0)),
            pl.BlockSpec((BH, BH), lambda i: (0, 0)),
        ],
        out_specs=pl.BlockSpec((_BN, BH), lambda i: (i, 0)),
        out_shape=jax.ShapeDtypeStruct((NP, BH), f32),
    )(p1, h1, dinv_b, b1t, W2bd)

    p2 = _mp(src4d, dst4d, h2)

    x2 = pl.pallas_call(
        _final_body,
        grid=(NP // _BN,),
        in_specs=[
            pl.BlockSpec((NC, _BN, BH), lambda i: (0, i, 0)),
            pl.BlockSpec((_BN, BH), lambda i: (i, 0)),
            pl.BlockSpec((_BN, BH), lambda i: (i, 0)),
            pl.BlockSpec((1, BH), lambda i: (0, 0)),
        ],
        out_specs=pl.BlockSpec((_BN, BH), lambda i: (i, 0)),
        out_shape=jax.ShapeDtypeStruct((NP, BH), f32),
    )(p2, h2, dinv_b, b2t)

    conv = jnp.transpose(x2[:N].reshape(N, B, H), (1, 0, 2)).reshape(B, N * H)

    Wp2p = jnp.pad(Wp2, ((0, 0), (0, 128 - NA)))
    bp2p = jnp.pad(bp2, (0, 128 - NA)).reshape(1, 128)
    Wv2p = jnp.pad(Wv2, ((0, 0), (0, 128 - 1)))
    bv2p = jnp.pad(bv2, (0, 128 - 1)).reshape(1, 128)

    pol_p, val_p = pl.pallas_call(
        _heads_body,
        grid=(_KS,),
        in_specs=[
            pl.BlockSpec((B, _BK), lambda k: (0, k)),
            pl.BlockSpec((_BK, 512), lambda k: (k, 0)),
            pl.BlockSpec((1, 512), lambda k: (0, 0)),
            pl.BlockSpec((512, 128), lambda k: (0, 0)),
            pl.BlockSpec((1, 128), lambda k: (0, 0)),
            pl.BlockSpec((_BK, 512), lambda k: (k, 0)),
            pl.BlockSpec((1, 512), lambda k: (0, 0)),
            pl.BlockSpec((512, 128), lambda k: (0, 0)),
            pl.BlockSpec((1, 128), lambda k: (0, 0)),
        ],
        out_specs=[
            pl.BlockSpec((B, 128), lambda k: (0, 0)),
            pl.BlockSpec((B, 128), lambda k: (0, 0)),
        ],
        out_shape=[
            jax.ShapeDtypeStruct((B, 128), f32),
            jax.ShapeDtypeStruct((B, 128), f32),
        ],
        scratch_shapes=[
            pltpu.VMEM((B, 512), f32),
            pltpu.VMEM((B, 512), f32),
        ],
        compiler_params=pltpu.CompilerParams(
            dimension_semantics=("arbitrary",)),
    )(conv, Wp1, bp1.reshape(1, 512), Wp2p, bp2p,
      Wv1, bv1.reshape(1, 512), Wv2p, bv2p)

    policy = pol_p[:, :NA]
    value = val_p[:, :1]
    return (policy, value)
